# baseline (speedup 1.0000x reference)
import jax
import jax.numpy as jnp
from jax import lax
from jax.experimental import pallas as pl
from jax.experimental.pallas import tpu as pltpu
from jax.experimental.pallas import tpu_sc as plsc

B = 16384
D = 64
PW = 128
LANES = 16
NUM_WORKERS = 32
BPW = B // NUM_WORKERS
GCHUNK = 128
NCHUNKS = BPW // GCHUNK


def _body(users_hbm, items_hbm, utab_hbm, itab_hbm, out_hbm,
          idx_u, idx_v, lidx_u, lidx_v, rows_u, rows_v, out_v, sem):
  wid = lax.axis_index("s") * 2 + lax.axis_index("c")
  base = wid * BPW

  for c in range(NCHUNKS):
    pltpu.sync_copy(users_hbm.at[pl.ds(base + c * GCHUNK, GCHUNK)], idx_u.at[c])
    pltpu.sync_copy(items_hbm.at[pl.ds(base + c * GCHUNK, GCHUNK)], idx_v.at[c])

  def shift_body(i, _):
    sl = pl.ds(i * LANES, LANES)
    c = i // (GCHUNK // LANES)
    j = i % (GCHUNK // LANES)
    csl = pl.ds(j * LANES, LANES)
    lidx_u[c, csl] = lax.shift_right_logical(idx_u[c, csl], 1)
    lidx_v[c, csl] = lax.shift_right_logical(idx_v[c, csl], 1)
    return 0

  for i in range(BPW // LANES):
    shift_body(i, 0)

  lane = lax.iota(jnp.int32, LANES)

  for c in range(NCHUNKS):
    cp_u = pltpu.make_async_copy(utab_hbm.at[lidx_u.at[c]], rows_u, sem)
    cp_v = pltpu.make_async_copy(itab_hbm.at[lidx_v.at[c]], rows_v, sem)
    cp_u.start()
    cp_v.start()
    cp_u.wait()
    cp_v.wait()

    def group_body(g, _):
      gsl = pl.ds(g * LANES, LANES)
      ridx = g * LANES + lane
      off_u = jnp.bitwise_and(idx_u[c, gsl], 1) * D
      off_v = jnp.bitwise_and(idx_v[c, gsl], 1) * D
      acc = (plsc.load_gather(rows_u, [ridx, off_u])
             * plsc.load_gather(rows_v, [ridx, off_v]))
      for d in range(1, D):
        acc = acc + (plsc.load_gather(rows_u, [ridx, off_u + d])
                     * plsc.load_gather(rows_v, [ridx, off_v + d]))
      out_v[pl.ds(c * GCHUNK + g * LANES, LANES)] = 1.0 / (1.0 + jnp.exp(-acc))
      return 0

    lax.fori_loop(0, GCHUNK // LANES, group_body, 0)

  pltpu.sync_copy(out_v, out_hbm.at[pl.ds(base, BPW)])


@jax.jit
def kernel(users, items, user_table, item_table):
  mesh = plsc.VectorSubcoreMesh(core_axis_name="c", subcore_axis_name="s")
  run = pl.kernel(
      _body,
      out_type=jax.ShapeDtypeStruct((B,), jnp.float32),
      mesh=mesh,
      scratch_types=[
          pltpu.VMEM((NCHUNKS, GCHUNK), jnp.int32),
          pltpu.VMEM((NCHUNKS, GCHUNK), jnp.int32),
          pltpu.VMEM((NCHUNKS, GCHUNK), jnp.int32),
          pltpu.VMEM((NCHUNKS, GCHUNK), jnp.int32),
          pltpu.VMEM((GCHUNK, PW), jnp.float32),
          pltpu.VMEM((GCHUNK, PW), jnp.float32),
          pltpu.VMEM((BPW,), jnp.float32),
          pltpu.SemaphoreType.DMA,
      ],
      compiler_params=pltpu.CompilerParams(
          needs_layout_passes=False,
      ),
  )
  ut2 = jnp.concatenate([user_table[0::2], user_table[1::2]], axis=1)
  it2 = jnp.concatenate([item_table[0::2], item_table[1::2]], axis=1)
  return run(users, items, ut2, it2)


# speedup vs baseline: 15.3486x; 15.3486x over previous
import jax
import jax.numpy as jnp
from jax import lax
from jax.experimental import pallas as pl
from jax.experimental.pallas import tpu as pltpu
from jax.experimental.pallas import tpu_sc as plsc

B = 16384
D = 64
PW = 128
LANES = 16
NUM_WORKERS = 32
BPW = B // NUM_WORKERS
GCHUNK = 128
NCHUNKS = BPW // GCHUNK


def _body(users_hbm, items_hbm, utab_hbm, itab_hbm, out_hbm,
          idx_u, idx_v, lidx_u, lidx_v, rows_u, rows_v, out_v, sem):
  wid = lax.axis_index("s") * 2 + lax.axis_index("c")
  base = wid * BPW

  for c in range(NCHUNKS):
    pltpu.sync_copy(users_hbm.at[pl.ds(base + c * GCHUNK, GCHUNK)], idx_u.at[c])
    pltpu.sync_copy(items_hbm.at[pl.ds(base + c * GCHUNK, GCHUNK)], idx_v.at[c])

  def shift_body(i, _):
    sl = pl.ds(i * LANES, LANES)
    c = i // (GCHUNK // LANES)
    j = i % (GCHUNK // LANES)
    csl = pl.ds(j * LANES, LANES)
    lidx_u[c, csl] = lax.shift_right_logical(idx_u[c, csl], 1)
    lidx_v[c, csl] = lax.shift_right_logical(idx_v[c, csl], 1)
    return 0

  for i in range(BPW // LANES):
    shift_body(i, 0)

  lane = lax.iota(jnp.int32, LANES)

  for c in range(NCHUNKS):
    cp_u = pltpu.make_async_copy(utab_hbm.at[lidx_u.at[c]], rows_u, sem)
    cp_v = pltpu.make_async_copy(itab_hbm.at[lidx_v.at[c]], rows_v, sem)
    cp_u.start()
    cp_v.start()
    cp_u.wait()
    cp_v.wait()

    def group_body(g, _):
      gsl = pl.ds(g * LANES, LANES)
      ridx = g * LANES + lane
      off_u = jnp.bitwise_and(idx_u[c, gsl], 1) * D
      off_v = jnp.bitwise_and(idx_v[c, gsl], 1) * D
      acc = (plsc.load_gather(rows_u, [ridx, off_u])
             * plsc.load_gather(rows_v, [ridx, off_v]))
      for d in range(1, D):
        acc = acc + (plsc.load_gather(rows_u, [ridx, off_u + d])
                     * plsc.load_gather(rows_v, [ridx, off_v + d]))
      out_v[pl.ds(c * GCHUNK + g * LANES, LANES)] = 1.0 / (1.0 + jnp.exp(-acc))
      return 0

    lax.fori_loop(0, GCHUNK // LANES, group_body, 0)

  pltpu.sync_copy(out_v, out_hbm.at[pl.ds(base, BPW)])


@jax.jit
def kernel(users, items, user_table, item_table):
  mesh = plsc.VectorSubcoreMesh(core_axis_name="c", subcore_axis_name="s")
  run = pl.kernel(
      _body,
      out_type=jax.ShapeDtypeStruct((B,), jnp.float32),
      mesh=mesh,
      scratch_types=[
          pltpu.VMEM((NCHUNKS, GCHUNK), jnp.int32),
          pltpu.VMEM((NCHUNKS, GCHUNK), jnp.int32),
          pltpu.VMEM((NCHUNKS, GCHUNK), jnp.int32),
          pltpu.VMEM((NCHUNKS, GCHUNK), jnp.int32),
          pltpu.VMEM((GCHUNK, PW), jnp.float32),
          pltpu.VMEM((GCHUNK, PW), jnp.float32),
          pltpu.VMEM((BPW,), jnp.float32),
          pltpu.SemaphoreType.DMA,
      ],
      compiler_params=pltpu.CompilerParams(
          needs_layout_passes=False,
      ),
  )
  ut2 = user_table.reshape(user_table.shape[0] // 2, PW)
  it2 = item_table.reshape(item_table.shape[0] // 2, PW)
  return run(users, items, ut2, it2)


# speedup vs baseline: 15.4665x; 1.0077x over previous
import jax
import jax.numpy as jnp
from jax import lax
from jax.experimental import pallas as pl
from jax.experimental.pallas import tpu as pltpu
from jax.experimental.pallas import tpu_sc as plsc

B = 16384
D = 64
LANES = 16
NUM_WORKERS = 32
BPW = B // NUM_WORKERS
GCHUNK = 128
NCHUNKS = BPW // GCHUNK


def _body(users_hbm, items_hbm, utab_hbm, itab_hbm, out_hbm,
          idx_u, idx_v, rows_u, rows_v, out_v, sem):
  wid = lax.axis_index("s") * 2 + lax.axis_index("c")
  base = wid * BPW

  pltpu.sync_copy(users_hbm.at[pl.ds(base, BPW)], idx_u)
  pltpu.sync_copy(items_hbm.at[pl.ds(base, BPW)], idx_v)

  copies = []
  for j in range(NCHUNKS):
    sl = pl.ds(j * GCHUNK, GCHUNK)
    copies.append(pltpu.make_async_copy(
        utab_hbm.at[idx_u.at[sl]], rows_u.at[sl], sem))
    copies.append(pltpu.make_async_copy(
        itab_hbm.at[idx_v.at[sl]], rows_v.at[sl], sem))
  for c in copies:
    c.start()
  for c in copies:
    c.wait()

  lane = lax.iota(jnp.int32, LANES)

  def group_body(g, _):
    ridx = g * LANES + lane
    acc = jnp.zeros((LANES,), jnp.float32)
    for d in range(D):
      cidx = jnp.full((LANES,), d, jnp.int32)
      acc = acc + (plsc.load_gather(rows_u, [ridx, cidx])
                   * plsc.load_gather(rows_v, [ridx, cidx]))
    out_v[pl.ds(g * LANES, LANES)] = 1.0 / (1.0 + jnp.exp(-acc))
    return 0

  lax.fori_loop(0, BPW // LANES, group_body, 0)

  pltpu.sync_copy(out_v, out_hbm.at[pl.ds(base, BPW)])


@jax.jit
def kernel(users, items, user_table, item_table):
  mesh = plsc.VectorSubcoreMesh(core_axis_name="c", subcore_axis_name="s")
  run = pl.kernel(
      _body,
      out_type=jax.ShapeDtypeStruct((B,), jnp.float32),
      mesh=mesh,
      scratch_types=[
          pltpu.VMEM((BPW,), jnp.int32),
          pltpu.VMEM((BPW,), jnp.int32),
          pltpu.VMEM((BPW, D), jnp.float32),
          pltpu.VMEM((BPW, D), jnp.float32),
          pltpu.VMEM((BPW,), jnp.float32),
          pltpu.SemaphoreType.DMA,
      ],
      compiler_params=pltpu.CompilerParams(
          use_tc_tiling_on_sc=False,
          needs_layout_passes=False,
      ),
  )
  return run(users, items, user_table, item_table)
